# Initial kernel scaffold; baseline (speedup 1.0000x reference)
import jax
import jax.numpy as jnp
from jax.experimental import pallas as pl


def kernel(uid_idx, iid_idx, seq_idx, mask, dense, user_table, item_table):
    raise NotImplementedError("write your pallas kernel here")


# speedup vs baseline: 1.0738x; 1.0738x over previous
import functools

import jax
import jax.numpy as jnp
from jax import lax
from jax.experimental import pallas as pl
from jax.experimental.pallas import tpu as pltpu
from jax.experimental.pallas import tpu_sc as plsc

B = 4096
L = 200
D = 32

_NC = 2
_NS = 16
_NW = _NC * _NS

_PER_W_B = B // _NW
_SEQ_TOTAL = B * L
_PER_W_SEQ = _SEQ_TOTAL // _NW
_CH = 1600
_NCH = _PER_W_SEQ // _CH


def _din_body(uid_hbm, iid_hbm, seq_hbm, ut_hbm, it_hbm,
              user_out, item_out, seq_out,
              idx_s, rows_s, idx_b, rows_b, sem):
    c = lax.axis_index("c")
    s = lax.axis_index("s")
    wid = s * _NC + c

    ub = wid * _PER_W_B
    pltpu.sync_copy(uid_hbm.at[pl.ds(ub, _PER_W_B)], idx_s)
    pltpu.async_copy(ut_hbm.at[idx_s], rows_s, sem).wait()
    pltpu.sync_copy(rows_s, user_out.at[pl.ds(ub, _PER_W_B)])

    pltpu.sync_copy(iid_hbm.at[pl.ds(ub, _PER_W_B)], idx_s)
    pltpu.async_copy(it_hbm.at[idx_s], rows_s, sem).wait()
    pltpu.sync_copy(rows_s, item_out.at[pl.ds(ub, _PER_W_B)])

    def chunk(j, carry):
        base = wid * _PER_W_SEQ + j * _CH
        pltpu.sync_copy(seq_hbm.at[pl.ds(base, _CH)], idx_b)
        pltpu.async_copy(it_hbm.at[idx_b], rows_b, sem).wait()
        pltpu.sync_copy(rows_b, seq_out.at[pl.ds(base, _CH)])
        return carry

    lax.fori_loop(0, _NCH, chunk, 0)


@jax.jit
def _din_sc(uid_idx, iid_idx, seq_flat, user_table, item_table):
    mesh = plsc.VectorSubcoreMesh(core_axis_name="c", subcore_axis_name="s")
    f = pl.kernel(
        _din_body,
        out_type=(
            jax.ShapeDtypeStruct((B, D), jnp.float32),
            jax.ShapeDtypeStruct((B, D), jnp.float32),
            jax.ShapeDtypeStruct((_SEQ_TOTAL, D), jnp.float32),
        ),
        mesh=mesh,
        compiler_params=pltpu.CompilerParams(use_tc_tiling_on_sc=False),
        scratch_types=[
            pltpu.VMEM((_PER_W_B,), jnp.int32),
            pltpu.VMEM((_PER_W_B, D), jnp.float32),
            pltpu.VMEM((_CH,), jnp.int32),
            pltpu.VMEM((_CH, D), jnp.float32),
            pltpu.SemaphoreType.DMA,
        ],
    )
    return f(uid_idx, iid_idx, seq_flat, user_table, item_table)


def kernel(uid_idx, iid_idx, seq_idx, mask, dense, user_table, item_table):
    del mask, dense
    seq_flat = seq_idx.reshape(_SEQ_TOTAL).astype(jnp.int32)
    user_embed, item_embed, seq_embed = _din_sc(
        uid_idx.astype(jnp.int32), iid_idx.astype(jnp.int32), seq_flat,
        user_table, item_table)
    return (user_embed, item_embed, seq_embed.reshape(B, L, D))


# speedup vs baseline: 1.0835x; 1.0090x over previous
import jax
import jax.numpy as jnp
from jax import lax
from jax.experimental import pallas as pl
from jax.experimental.pallas import tpu as pltpu
from jax.experimental.pallas import tpu_sc as plsc

B = 4096
L = 200
D = 32

_NC = 2
_NS = 16
_NW = _NC * _NS

_PER_W_B = B // _NW
_SEQ_TOTAL = B * L
_PER_W_SEQ = _SEQ_TOTAL // _NW
_CH = 1600
_NCH = _PER_W_SEQ // _CH


def _din_body(uid_hbm, iid_hbm, seq_hbm, ut_hbm, it_hbm,
              user_out, item_out, seq_out,
              uidx_v, urows_v, iidx_v, irows_v,
              idx0, idx1, rows0, rows1,
              usem, isem, g0, g1, o0, o1):
    c = lax.axis_index("c")
    s = lax.axis_index("s")
    wid = s * _NC + c
    ub = wid * _PER_W_B
    sbase = wid * _PER_W_SEQ

    pltpu.sync_copy(uid_hbm.at[pl.ds(ub, _PER_W_B)], uidx_v)
    ucopy = pltpu.make_async_copy(ut_hbm.at[uidx_v], urows_v, usem)
    ucopy.start()
    pltpu.sync_copy(iid_hbm.at[pl.ds(ub, _PER_W_B)], iidx_v)
    icopy = pltpu.make_async_copy(it_hbm.at[iidx_v], irows_v, isem)
    icopy.start()

    idx = (idx0, idx1)
    rows = (rows0, rows1)
    gsem = (g0, g1)
    osem = (o0, o1)

    def load_and_gather(j, b):
        pltpu.sync_copy(seq_hbm.at[pl.ds(sbase + j * _CH, _CH)], idx[b])
        pltpu.make_async_copy(it_hbm.at[idx[b]], rows[b], gsem[b]).start()

    def writeback(j, b):
        return pltpu.make_async_copy(
            rows[b], seq_out.at[pl.ds(sbase + j * _CH, _CH)], osem[b])

    load_and_gather(0, 0)
    load_and_gather(1, 1)

    def outer(jj, carry):
        for b in range(2):
            j = 2 * jj + b
            pltpu.make_async_copy(it_hbm.at[idx[b]], rows[b], gsem[b]).wait()
            wb = writeback(j, b)
            wb.start()
            wb.wait()
            load_and_gather(j + 2, b)
        return carry

    lax.fori_loop(0, (_NCH - 2) // 2, outer, 0)

    pltpu.make_async_copy(it_hbm.at[idx[0]], rows[0], gsem[0]).wait()
    writeback(_NCH - 2, 0).start()
    pltpu.make_async_copy(it_hbm.at[idx[1]], rows[1], gsem[1]).wait()
    writeback(_NCH - 1, 1).start()
    writeback(_NCH - 2, 0).wait()
    writeback(_NCH - 1, 1).wait()

    ucopy.wait()
    pltpu.sync_copy(urows_v, user_out.at[pl.ds(ub, _PER_W_B)])
    icopy.wait()
    pltpu.sync_copy(irows_v, item_out.at[pl.ds(ub, _PER_W_B)])


@jax.jit
def _din_sc(uid_idx, iid_idx, seq_flat, user_table, item_table):
    mesh = plsc.VectorSubcoreMesh(core_axis_name="c", subcore_axis_name="s")
    f = pl.kernel(
        _din_body,
        out_type=(
            jax.ShapeDtypeStruct((B, D), jnp.float32),
            jax.ShapeDtypeStruct((B, D), jnp.float32),
            jax.ShapeDtypeStruct((_SEQ_TOTAL, D), jnp.float32),
        ),
        mesh=mesh,
        compiler_params=pltpu.CompilerParams(use_tc_tiling_on_sc=False),
        scratch_types=[
            pltpu.VMEM((_PER_W_B,), jnp.int32),
            pltpu.VMEM((_PER_W_B, D), jnp.float32),
            pltpu.VMEM((_PER_W_B,), jnp.int32),
            pltpu.VMEM((_PER_W_B, D), jnp.float32),
            pltpu.VMEM((_CH,), jnp.int32),
            pltpu.VMEM((_CH,), jnp.int32),
            pltpu.VMEM((_CH, D), jnp.float32),
            pltpu.VMEM((_CH, D), jnp.float32),
            pltpu.SemaphoreType.DMA,
            pltpu.SemaphoreType.DMA,
            pltpu.SemaphoreType.DMA,
            pltpu.SemaphoreType.DMA,
            pltpu.SemaphoreType.DMA,
            pltpu.SemaphoreType.DMA,
        ],
    )
    return f(uid_idx, iid_idx, seq_flat, user_table, item_table)


def kernel(uid_idx, iid_idx, seq_idx, mask, dense, user_table, item_table):
    del mask, dense
    seq_flat = seq_idx.reshape(_SEQ_TOTAL).astype(jnp.int32)
    user_embed, item_embed, seq_embed = _din_sc(
        uid_idx.astype(jnp.int32), iid_idx.astype(jnp.int32), seq_flat,
        user_table, item_table)
    return (user_embed, item_embed, seq_embed.reshape(B, L, D))
